# Initial kernel scaffold; baseline (speedup 1.0000x reference)
import jax
import jax.numpy as jnp
from jax.experimental import pallas as pl


def kernel(x, w_in, codebooks, w_out):
    raise NotImplementedError("write your pallas kernel here")


# speedup vs baseline: 1.2540x; 1.2540x over previous
import functools

import jax
import jax.numpy as jnp
from jax import lax
from jax.experimental import pallas as pl
from jax.experimental.pallas import tpu as pltpu
from jax.experimental.pallas import tpu_sc as plsc

NUM_Q = 4
HID = 1024
CB = 8192
SUB = HID // NUM_Q
EPS = 1e-05
B, T = 4, 2048
BT = B * T

TM = 128
NT = BT // TM

NC, NS, L = 2, 16, 16
NW = NC * NS
ROWS_TOTAL = BT * NUM_Q
ROWS_PER_W = ROWS_TOTAL // NW
CHUNK = 128
N_CHUNKS = ROWS_PER_W // CHUNK

TM_NORM = 512


def _dist_body(x_ref, xq_ref, wq_ref, cb_ref, idx_ref, c2_ref):
    t = pl.program_id(1)
    cb = cb_ref[0]

    @pl.when(t == 0)
    def _():
        c2_ref[...] = jnp.sum(cb * cb, axis=1, keepdims=True)

    x = x_ref[...]
    v = jnp.mean(x * x, axis=1, keepdims=True)
    hq = xq_ref[...] * lax.rsqrt(v + EPS) * wq_ref[...]
    xc = lax.dot_general(cb, hq, (((1,), (1,)), ((), ())),
                         preferred_element_type=jnp.float32,
                         precision=lax.Precision.DEFAULT)
    scores = c2_ref[...] - 2.0 * xc
    m = jnp.min(scores, axis=0, keepdims=True)
    rows = lax.broadcasted_iota(jnp.int32, scores.shape, 0)
    idx = jnp.min(jnp.where(scores == m, rows, CB), axis=0)
    idx_ref[0, 0, :] = idx


def _dist(x2d, w_in2d, codebooks):
    return pl.pallas_call(
        _dist_body,
        grid=(NUM_Q, NT),
        in_specs=[
            pl.BlockSpec((TM, HID), lambda q, t: (t, 0)),
            pl.BlockSpec((TM, SUB), lambda q, t: (t, q)),
            pl.BlockSpec((1, SUB), lambda q, t: (0, q)),
            pl.BlockSpec((1, CB, SUB), lambda q, t: (q, 0, 0)),
        ],
        out_specs=pl.BlockSpec((1, 1, TM), lambda q, t: (q, 0, t)),
        out_shape=jax.ShapeDtypeStruct((NUM_Q, 1, BT), jnp.int32),
        scratch_shapes=[pltpu.VMEM((CB, 1), jnp.float32)],
    )(x2d, x2d, w_in2d, codebooks)


def _sc_gather_body(table_hbm, idx_hbm, out_hbm, idx_v, rows_v, sem):
    wid = lax.axis_index("s") * NC + lax.axis_index("c")
    base = wid * ROWS_PER_W
    offs = (lax.iota(jnp.int32, L) & (NUM_Q - 1)) * CB
    for c in range(N_CHUNKS):
        off = base + c * CHUNK
        pltpu.sync_copy(idx_hbm.at[pl.ds(off, CHUNK)], idx_v)
        for j in range(CHUNK // L):
            sl = pl.ds(j * L, L)
            idx_v[sl] = idx_v[sl] + offs
        pltpu.async_copy(table_hbm.at[idx_v], rows_v, sem).wait()
        pltpu.sync_copy(rows_v, out_hbm.at[pl.ds(off, CHUNK)])


def _sc_gather(table, idx_flat):
    mesh = plsc.VectorSubcoreMesh(core_axis_name="c", subcore_axis_name="s")
    k = functools.partial(
        pl.kernel,
        mesh=mesh,
        out_type=jax.ShapeDtypeStruct((ROWS_TOTAL, SUB), jnp.float32),
        scratch_types=[
            pltpu.VMEM((CHUNK,), jnp.int32),
            pltpu.VMEM((CHUNK, SUB), jnp.float32),
            pltpu.SemaphoreType.DMA,
        ],
    )(_sc_gather_body)
    return k(table, idx_flat)


def _outnorm_body(qz_ref, w_ref, o_ref):
    qv = qz_ref[...]
    v = jnp.mean(qv * qv, axis=1, keepdims=True)
    o_ref[...] = qv * lax.rsqrt(v + EPS) * w_ref[...]


def _outnorm(quantized, w_out2d):
    return pl.pallas_call(
        _outnorm_body,
        grid=(BT // TM_NORM,),
        in_specs=[
            pl.BlockSpec((TM_NORM, HID), lambda t: (t, 0)),
            pl.BlockSpec((1, HID), lambda t: (0, 0)),
        ],
        out_specs=pl.BlockSpec((TM_NORM, HID), lambda t: (t, 0)),
        out_shape=jax.ShapeDtypeStruct((BT, HID), jnp.float32),
    )(quantized, w_out2d)


def kernel(x, w_in, codebooks, w_out):
    x2d = x.reshape(BT, HID)
    w_in2d = w_in.reshape(1, HID)
    w_out2d = w_out.reshape(1, HID)

    idx_qt = _dist(x2d, w_in2d, codebooks).reshape(NUM_Q, BT)
    idx_tq = idx_qt.T
    idx_flat = idx_tq.reshape(ROWS_TOTAL)

    table = codebooks.reshape(NUM_Q * CB, SUB)
    rows = _sc_gather(table, idx_flat)

    out2d = _outnorm(rows.reshape(BT, HID), w_out2d)
    return out2d.reshape(B, T, HID), idx_tq.reshape(B, T, NUM_Q)


# speedup vs baseline: 3.1147x; 2.4838x over previous
import functools

import jax
import jax.numpy as jnp
from jax import lax
from jax.experimental import pallas as pl
from jax.experimental.pallas import tpu as pltpu
from jax.experimental.pallas import tpu_sc as plsc

NUM_Q = 4
HID = 1024
CB = 8192
SUB = HID // NUM_Q
EPS = 1e-05
B, T = 4, 2048
BT = B * T

TM = 512
NT = BT // TM
KC = 2048
NKC = CB // KC

NC, NS, L = 2, 16, 16
NW = NC * NS
ROWS_TOTAL = BT * NUM_Q
ROWS_PER_W = ROWS_TOTAL // NW
CHUNK = 128
N_CHUNKS = ROWS_PER_W // CHUNK

TM_NORM = 512


def _dist_body(hq_ref, cb_ref, idx_ref, c2_ref):
    t = pl.program_id(1)

    @pl.when(t == 0)
    def _():
        for kc in range(NKC):
            cbc = cb_ref[0, kc * KC:(kc + 1) * KC, :]
            c2_ref[kc * KC:(kc + 1) * KC, :] = jnp.sum(cbc * cbc, axis=1,
                                                       keepdims=True)

    hqm2 = hq_ref[...] * -2.0
    best = None
    bidx = None
    for kc in range(NKC):
        cbc = cb_ref[0, kc * KC:(kc + 1) * KC, :]
        xc = lax.dot_general(cbc, hqm2, (((1,), (1,)), ((), ())),
                             preferred_element_type=jnp.float32,
                             precision=lax.Precision.DEFAULT)
        scores = c2_ref[kc * KC:(kc + 1) * KC, :] + xc
        m = jnp.min(scores, axis=0, keepdims=True)
        ii = (jnp.argmin(scores, axis=0).astype(jnp.int32)
              + kc * KC).reshape(1, TM)
        if best is None:
            best, bidx = m, ii
        else:
            upd = m < best
            bidx = jnp.where(upd, ii, bidx)
            best = jnp.where(upd, m, best)
    idx_ref[0, 0, :] = bidx[0]


def _dist(h2d, codebooks):
    return pl.pallas_call(
        _dist_body,
        grid=(NUM_Q, NT),
        in_specs=[
            pl.BlockSpec((TM, SUB), lambda q, t: (t, q)),
            pl.BlockSpec((1, CB, SUB), lambda q, t: (q, 0, 0)),
        ],
        out_specs=pl.BlockSpec((1, 1, TM), lambda q, t: (q, 0, t)),
        out_shape=jax.ShapeDtypeStruct((NUM_Q, 1, BT), jnp.int32),
        scratch_shapes=[pltpu.VMEM((CB, 1), jnp.float32)],
    )(h2d, codebooks)


def _sc_gather_body(table_hbm, idx_hbm, out_hbm, idx_v, rows_v, sem):
    wid = lax.axis_index("s") * NC + lax.axis_index("c")
    base = wid * ROWS_PER_W
    offs = (lax.iota(jnp.int32, L) & (NUM_Q - 1)) * CB
    for c in range(N_CHUNKS):
        off = base + c * CHUNK
        pltpu.sync_copy(idx_hbm.at[pl.ds(off, CHUNK)], idx_v)
        for j in range(CHUNK // L):
            sl = pl.ds(j * L, L)
            idx_v[sl] = idx_v[sl] + offs
        pltpu.async_copy(table_hbm.at[idx_v], rows_v, sem).wait()
        pltpu.sync_copy(rows_v, out_hbm.at[pl.ds(off, CHUNK)])


def _sc_gather(table, idx_flat):
    mesh = plsc.VectorSubcoreMesh(core_axis_name="c", subcore_axis_name="s")
    k = functools.partial(
        pl.kernel,
        mesh=mesh,
        out_type=jax.ShapeDtypeStruct((ROWS_TOTAL, SUB), jnp.float32),
        scratch_types=[
            pltpu.VMEM((CHUNK,), jnp.int32),
            pltpu.VMEM((CHUNK, SUB), jnp.float32),
            pltpu.SemaphoreType.DMA,
        ],
    )(_sc_gather_body)
    return k(table, idx_flat)


def _rms_body(qz_ref, w_ref, o_ref):
    qv = qz_ref[...]
    v = jnp.mean(qv * qv, axis=1, keepdims=True)
    o_ref[...] = qv * lax.rsqrt(v + EPS) * w_ref[...]


def _rms(arr2d, w2d):
    return pl.pallas_call(
        _rms_body,
        grid=(BT // TM_NORM,),
        in_specs=[
            pl.BlockSpec((TM_NORM, HID), lambda t: (t, 0)),
            pl.BlockSpec((1, HID), lambda t: (0, 0)),
        ],
        out_specs=pl.BlockSpec((TM_NORM, HID), lambda t: (t, 0)),
        out_shape=jax.ShapeDtypeStruct((BT, HID), jnp.float32),
    )(arr2d, w2d)


def kernel(x, w_in, codebooks, w_out):
    x2d = x.reshape(BT, HID)
    w_in2d = w_in.reshape(1, HID)
    w_out2d = w_out.reshape(1, HID)

    h2d = _rms(x2d, w_in2d)
    idx_qt = _dist(h2d, codebooks).reshape(NUM_Q, BT)
    idx_tq = idx_qt.T
    idx_flat = idx_tq.reshape(ROWS_TOTAL)

    table = codebooks.reshape(NUM_Q * CB, SUB)
    rows = _sc_gather(table, idx_flat)

    out2d = _rms(rows.reshape(BT, HID), w_out2d)
    return out2d.reshape(B, T, HID), idx_tq.reshape(B, T, NUM_Q)
